# baseline (device time: 47260 ns/iter reference)
import jax
import jax.numpy as jnp
from jax import lax
from jax.experimental import pallas as pl
from jax.experimental.pallas import tpu as pltpu

CHUNKS = [(0, 256), (256, 384), (640, 512), (1152, 640),
          (1792, 768), (2560, 768), (3328, 512), (3840, 256)]
NC = len(CHUNKS)


def kernel(x, dy):
    m, d = x.shape
    _, f = dy.shape
    blk = d // 4
    assert sum(s for _, s in CHUNKS) == f

    def body(x_ref, dy_ref, out_ref,
             dyf, dy16, xbf, xaf, xsend, xrecv, ysend, yrecv,
             dy_sems, x_sems, xs_sems, xr_sems, ys_sems, yr_sems,
             oo_sems, ot_sems):
        my_x = lax.axis_index("x")
        my_y = lax.axis_index("y")
        b = 2 * my_x + my_y
        bx = 2 * (1 - my_x) + my_y

        xb_cp = pltpu.make_async_copy(
            x_ref.at[:, pl.ds(bx * blk, blk)], xbf, x_sems.at[0])
        xb_cp.start()
        xa_cp = pltpu.make_async_copy(
            x_ref.at[:, pl.ds(b * blk, blk)], xaf, x_sems.at[1])
        xa_cp.start()
        dy_copies = []
        for c, (off, sz) in enumerate(CHUNKS):
            sl = pl.ds(off, sz)
            cp = pltpu.make_async_copy(dy_ref.at[:, sl], dyf.at[:, sl],
                                       dy_sems.at[c])
            cp.start()
            dy_copies.append(cp)

        xb_cp.wait()
        xb = xbf[:, :].astype(jnp.bfloat16)

        barrier = pltpu.get_barrier_semaphore()
        for nbr in [(1 - my_x, my_y), (my_x, 1 - my_y)]:
            pl.semaphore_signal(
                barrier, inc=1,
                device_id=nbr, device_id_type=pl.DeviceIdType.MESH,
            )
        pl.semaphore_wait(barrier, 2)

        x_rdmas = []
        for c, (off, sz) in enumerate(CHUNKS):
            sl = pl.ds(off, sz)
            dy_copies[c].wait()
            dy16[:, sl] = dyf[:, sl].astype(jnp.bfloat16)
            xsend[:, sl] = lax.dot_general(
                xb, dy16[:, sl], (((0,), (0,)), ((), ())),
                preferred_element_type=jnp.float32,
            ).astype(jnp.bfloat16)
            rdma = pltpu.make_async_remote_copy(
                src_ref=xsend.at[:, sl], dst_ref=xrecv.at[:, sl],
                send_sem=xs_sems.at[c], recv_sem=xr_sems.at[c],
                device_id=(1 - my_x, my_y),
                device_id_type=pl.DeviceIdType.MESH,
            )
            rdma.start()
            x_rdmas.append(rdma)

        xa_cp.wait()
        xa = xaf[:, :].astype(jnp.bfloat16)
        A = lax.dot_general(
            xa, dy16[:, :], (((0,), (0,)), ((), ())),
            preferred_element_type=jnp.float32,
        )

        y_rdmas = []
        o_copies = []
        for c, (off, sz) in enumerate(CHUNKS):
            sl = pl.ds(off, sz)
            x_rdmas[c].wait_recv()
            S = A[:, off:off + sz] + xrecv[:, sl].astype(jnp.float32)
            ysend[:, sl] = S.astype(jnp.bfloat16)
            rdma = pltpu.make_async_remote_copy(
                src_ref=ysend.at[:, sl], dst_ref=yrecv.at[:, sl],
                send_sem=ys_sems.at[c], recv_sem=yr_sems.at[c],
                device_id=(my_x, 1 - my_y),
                device_id_type=pl.DeviceIdType.MESH,
            )
            rdma.start()
            y_rdmas.append(rdma)
            cp = pltpu.make_async_copy(
                ysend.at[:, sl],
                out_ref.at[pl.ds(my_y * blk, blk), sl],
                oo_sems.at[c],
            )
            cp.start()
            o_copies.append(cp)

        t_copies = []
        for c, (off, sz) in enumerate(CHUNKS):
            sl = pl.ds(off, sz)
            y_rdmas[c].wait_recv()
            cp = pltpu.make_async_copy(
                yrecv.at[:, sl],
                out_ref.at[pl.ds((1 - my_y) * blk, blk), sl],
                ot_sems.at[c],
            )
            cp.start()
            t_copies.append(cp)

        for c in range(NC):
            o_copies[c].wait()
            t_copies[c].wait()
            x_rdmas[c].wait_send()
            y_rdmas[c].wait_send()

    return pl.pallas_call(
        body,
        out_shape=jax.ShapeDtypeStruct((d // 2, f), jnp.bfloat16),
        in_specs=[
            pl.BlockSpec(memory_space=pltpu.MemorySpace.HBM),
            pl.BlockSpec(memory_space=pltpu.MemorySpace.HBM),
        ],
        out_specs=pl.BlockSpec(memory_space=pltpu.MemorySpace.HBM),
        scratch_shapes=[
            pltpu.VMEM((m, f), jnp.float32),
            pltpu.VMEM((m, f), jnp.bfloat16),
            pltpu.VMEM((m, blk), jnp.float32),
            pltpu.VMEM((m, blk), jnp.float32),
            pltpu.VMEM((blk, f), jnp.bfloat16),
            pltpu.VMEM((blk, f), jnp.bfloat16),
            pltpu.VMEM((blk, f), jnp.bfloat16),
            pltpu.VMEM((blk, f), jnp.bfloat16),
            pltpu.SemaphoreType.DMA((NC,)),
            pltpu.SemaphoreType.DMA((2,)),
            pltpu.SemaphoreType.DMA((NC,)),
            pltpu.SemaphoreType.DMA((NC,)),
            pltpu.SemaphoreType.DMA((NC,)),
            pltpu.SemaphoreType.DMA((NC,)),
            pltpu.SemaphoreType.DMA((NC,)),
            pltpu.SemaphoreType.DMA((NC,)),
        ],
        compiler_params=pltpu.CompilerParams(
            collective_id=0, vmem_limit_bytes=100 * 1024 * 1024,
        ),
    )(x, dy)


# device time: 44365 ns/iter; 1.0653x vs baseline; 1.0653x over previous
import jax
import jax.numpy as jnp
from jax import lax
from jax.experimental import pallas as pl
from jax.experimental.pallas import tpu as pltpu

CHUNKS = [(0, 256), (256, 384), (640, 512), (1152, 640),
          (1792, 768), (2560, 768), (3328, 512), (3840, 256)]
NC = len(CHUNKS)


def kernel(x, dy):
    m, d = x.shape
    _, f = dy.shape
    blk = d // 4
    assert sum(s for _, s in CHUNKS) == f

    def body(x_ref, dy_ref, out_ref,
             dyf, dy16, xsend, xrecv, ysend, yrecv,
             dy_sems, xs_sems, xr_sems, ys_sems, yr_sems,
             oo_sems, ot_sems):
        my_x = lax.axis_index("x")
        my_y = lax.axis_index("y")
        b = 2 * my_x + my_y
        bx = 2 * (1 - my_x) + my_y

        dy_copies = []
        for c, (off, sz) in enumerate(CHUNKS):
            sl = pl.ds(off, sz)
            cp = pltpu.make_async_copy(dy_ref.at[:, sl], dyf.at[:, sl],
                                       dy_sems.at[c])
            cp.start()
            dy_copies.append(cp)

        xb = x_ref[:, pl.ds(bx * blk, blk)].astype(jnp.bfloat16)
        xa = x_ref[:, pl.ds(b * blk, blk)].astype(jnp.bfloat16)

        barrier = pltpu.get_barrier_semaphore()
        for nbr in [(1 - my_x, my_y), (my_x, 1 - my_y)]:
            pl.semaphore_signal(
                barrier, inc=1,
                device_id=nbr, device_id_type=pl.DeviceIdType.MESH,
            )
        pl.semaphore_wait(barrier, 2)

        x_rdmas = []
        for c, (off, sz) in enumerate(CHUNKS):
            sl = pl.ds(off, sz)
            dy_copies[c].wait()
            dy16[:, sl] = dyf[:, sl].astype(jnp.bfloat16)
            xsend[:, sl] = lax.dot_general(
                xb, dy16[:, sl], (((0,), (0,)), ((), ())),
                preferred_element_type=jnp.float32,
            ).astype(jnp.bfloat16)
            rdma = pltpu.make_async_remote_copy(
                src_ref=xsend.at[:, sl], dst_ref=xrecv.at[:, sl],
                send_sem=xs_sems.at[c], recv_sem=xr_sems.at[c],
                device_id=(1 - my_x, my_y),
                device_id_type=pl.DeviceIdType.MESH,
            )
            rdma.start()
            x_rdmas.append(rdma)

        A = lax.dot_general(
            xa, dy16[:, :], (((0,), (0,)), ((), ())),
            preferred_element_type=jnp.float32,
        )

        y_rdmas = []
        o_copies = []
        for c, (off, sz) in enumerate(CHUNKS):
            sl = pl.ds(off, sz)
            x_rdmas[c].wait_recv()
            S = A[:, off:off + sz] + xrecv[:, sl].astype(jnp.float32)
            ysend[:, sl] = S.astype(jnp.bfloat16)
            rdma = pltpu.make_async_remote_copy(
                src_ref=ysend.at[:, sl], dst_ref=yrecv.at[:, sl],
                send_sem=ys_sems.at[c], recv_sem=yr_sems.at[c],
                device_id=(my_x, 1 - my_y),
                device_id_type=pl.DeviceIdType.MESH,
            )
            rdma.start()
            y_rdmas.append(rdma)
            cp = pltpu.make_async_copy(
                ysend.at[:, sl],
                out_ref.at[pl.ds(my_y * blk, blk), sl],
                oo_sems.at[c],
            )
            cp.start()
            o_copies.append(cp)

        t_copies = []
        for c, (off, sz) in enumerate(CHUNKS):
            sl = pl.ds(off, sz)
            y_rdmas[c].wait_recv()
            cp = pltpu.make_async_copy(
                yrecv.at[:, sl],
                out_ref.at[pl.ds((1 - my_y) * blk, blk), sl],
                ot_sems.at[c],
            )
            cp.start()
            t_copies.append(cp)

        for c in range(NC):
            o_copies[c].wait()
            t_copies[c].wait()
            x_rdmas[c].wait_send()
            y_rdmas[c].wait_send()

    return pl.pallas_call(
        body,
        out_shape=jax.ShapeDtypeStruct((d // 2, f), jnp.bfloat16),
        in_specs=[
            pl.BlockSpec(memory_space=pltpu.VMEM),
            pl.BlockSpec(memory_space=pltpu.MemorySpace.HBM),
        ],
        out_specs=pl.BlockSpec(memory_space=pltpu.MemorySpace.HBM),
        scratch_shapes=[
            pltpu.VMEM((m, f), jnp.float32),
            pltpu.VMEM((m, f), jnp.bfloat16),
            pltpu.VMEM((blk, f), jnp.bfloat16),
            pltpu.VMEM((blk, f), jnp.bfloat16),
            pltpu.VMEM((blk, f), jnp.bfloat16),
            pltpu.VMEM((blk, f), jnp.bfloat16),
            pltpu.SemaphoreType.DMA((NC,)),
            pltpu.SemaphoreType.DMA((NC,)),
            pltpu.SemaphoreType.DMA((NC,)),
            pltpu.SemaphoreType.DMA((NC,)),
            pltpu.SemaphoreType.DMA((NC,)),
            pltpu.SemaphoreType.DMA((NC,)),
            pltpu.SemaphoreType.DMA((NC,)),
        ],
        compiler_params=pltpu.CompilerParams(
            collective_id=0, vmem_limit_bytes=100 * 1024 * 1024,
        ),
    )(x, dy)


# device time: 44216 ns/iter; 1.0688x vs baseline; 1.0034x over previous
import jax
import jax.numpy as jnp
from jax import lax
from jax.experimental import pallas as pl
from jax.experimental.pallas import tpu as pltpu

CHUNKS = [(0, 256), (256, 384), (640, 512), (1152, 640),
          (1792, 768), (2560, 768), (3328, 512), (3840, 256)]
NC = len(CHUNKS)


def kernel(x, dy):
    m, d = x.shape
    _, f = dy.shape
    blk = d // 4
    assert sum(s for _, s in CHUNKS) == f

    def body(x_ref, dy_ref, out_ref,
             dyf, dy16, xsend, xrecv, ysend, yrecv,
             dy_sems, xs_sems, xr_sems, ys_sems, yr_sems,
             oo_sems, ot_sems):
        my_x = lax.axis_index("x")
        my_y = lax.axis_index("y")
        b = 2 * my_x + my_y
        bx = 2 * (1 - my_x) + my_y

        dy_copies = []
        for c, (off, sz) in enumerate(CHUNKS):
            sl = pl.ds(off, sz)
            cp = pltpu.make_async_copy(dy_ref.at[:, sl], dyf.at[:, sl],
                                       dy_sems.at[c])
            cp.start()
            dy_copies.append(cp)

        barrier = pltpu.get_barrier_semaphore()
        for nbr in [(1 - my_x, my_y), (my_x, 1 - my_y)]:
            pl.semaphore_signal(
                barrier, inc=1,
                device_id=nbr, device_id_type=pl.DeviceIdType.MESH,
            )

        xb = x_ref[:, pl.ds(bx * blk, blk)].astype(jnp.bfloat16)
        xa = x_ref[:, pl.ds(b * blk, blk)].astype(jnp.bfloat16)

        PRE = 2

        def head_compute(c):
            off, sz = CHUNKS[c]
            sl = pl.ds(off, sz)
            dy_copies[c].wait()
            dy16[:, sl] = dyf[:, sl].astype(jnp.bfloat16)
            xsend[:, sl] = lax.dot_general(
                xb, dy16[:, sl], (((0,), (0,)), ((), ())),
                preferred_element_type=jnp.float32,
            ).astype(jnp.bfloat16)

        def head_send(c):
            off, sz = CHUNKS[c]
            sl = pl.ds(off, sz)
            rdma = pltpu.make_async_remote_copy(
                src_ref=xsend.at[:, sl], dst_ref=xrecv.at[:, sl],
                send_sem=xs_sems.at[c], recv_sem=xr_sems.at[c],
                device_id=(1 - my_x, my_y),
                device_id_type=pl.DeviceIdType.MESH,
            )
            rdma.start()
            return rdma

        for c in range(PRE):
            head_compute(c)
        pl.semaphore_wait(barrier, 2)

        x_rdmas = [head_send(c) for c in range(PRE)]
        for c in range(PRE, NC):
            head_compute(c)
            x_rdmas.append(head_send(c))

        A = lax.dot_general(
            xa, dy16[:, :], (((0,), (0,)), ((), ())),
            preferred_element_type=jnp.float32,
        )

        y_rdmas = []
        o_copies = []
        for c, (off, sz) in enumerate(CHUNKS):
            sl = pl.ds(off, sz)
            x_rdmas[c].wait_recv()
            S = A[:, off:off + sz] + xrecv[:, sl].astype(jnp.float32)
            ysend[:, sl] = S.astype(jnp.bfloat16)
            rdma = pltpu.make_async_remote_copy(
                src_ref=ysend.at[:, sl], dst_ref=yrecv.at[:, sl],
                send_sem=ys_sems.at[c], recv_sem=yr_sems.at[c],
                device_id=(my_x, 1 - my_y),
                device_id_type=pl.DeviceIdType.MESH,
            )
            rdma.start()
            y_rdmas.append(rdma)
            cp = pltpu.make_async_copy(
                ysend.at[:, sl],
                out_ref.at[pl.ds(my_y * blk, blk), sl],
                oo_sems.at[c],
            )
            cp.start()
            o_copies.append(cp)

        t_copies = []
        for c, (off, sz) in enumerate(CHUNKS):
            sl = pl.ds(off, sz)
            y_rdmas[c].wait_recv()
            cp = pltpu.make_async_copy(
                yrecv.at[:, sl],
                out_ref.at[pl.ds((1 - my_y) * blk, blk), sl],
                ot_sems.at[c],
            )
            cp.start()
            t_copies.append(cp)

        for c in range(NC):
            o_copies[c].wait()
            t_copies[c].wait()
            x_rdmas[c].wait_send()
            y_rdmas[c].wait_send()

    return pl.pallas_call(
        body,
        out_shape=jax.ShapeDtypeStruct((d // 2, f), jnp.bfloat16),
        in_specs=[
            pl.BlockSpec(memory_space=pltpu.VMEM),
            pl.BlockSpec(memory_space=pltpu.MemorySpace.HBM),
        ],
        out_specs=pl.BlockSpec(memory_space=pltpu.MemorySpace.HBM),
        scratch_shapes=[
            pltpu.VMEM((m, f), jnp.float32),
            pltpu.VMEM((m, f), jnp.bfloat16),
            pltpu.VMEM((blk, f), jnp.bfloat16),
            pltpu.VMEM((blk, f), jnp.bfloat16),
            pltpu.VMEM((blk, f), jnp.bfloat16),
            pltpu.VMEM((blk, f), jnp.bfloat16),
            pltpu.SemaphoreType.DMA((NC,)),
            pltpu.SemaphoreType.DMA((NC,)),
            pltpu.SemaphoreType.DMA((NC,)),
            pltpu.SemaphoreType.DMA((NC,)),
            pltpu.SemaphoreType.DMA((NC,)),
            pltpu.SemaphoreType.DMA((NC,)),
            pltpu.SemaphoreType.DMA((NC,)),
        ],
        compiler_params=pltpu.CompilerParams(
            collective_id=0, vmem_limit_bytes=100 * 1024 * 1024,
        ),
    )(x, dy)
